# Initial kernel scaffold; baseline (speedup 1.0000x reference)
import jax
import jax.numpy as jnp
from jax.experimental import pallas as pl


def kernel(x, tgt_pad, Wg, w1, w2, w3):
    raise NotImplementedError("write your pallas kernel here")


# speedup vs baseline: 1.4459x; 1.4459x over previous
import functools

import jax
import jax.numpy as jnp
from jax.experimental import pallas as pl
from jax.experimental.pallas import tpu as pltpu

_B, _S, _DIM = 4, 2048, 1024
_E, _TOPK, _HID = 8, 2, 2048
_N = _B * _S
_T = 256
_NB = (_TOPK * _N) // _T + _E
_NBT = _NB * _T
_RT = 1024


def _router_body(x_ref, wg_ref, sel0_ref, sel1_ref, w0_ref, w1_ref):
    xb = x_ref[...]
    wg = wg_ref[...]
    logits = jax.lax.dot_general(
        xb, wg, (((1,), (1,)), ((), ())),
        preferred_element_type=jnp.float32)
    iota = jax.lax.broadcasted_iota(jnp.int32, logits.shape, 1)
    l0 = jnp.max(logits, axis=1, keepdims=True)
    a0 = jnp.min(jnp.where(logits == l0, iota, _E), axis=1, keepdims=True)
    masked = jnp.where(iota == a0, -jnp.inf, logits)
    l1 = jnp.max(masked, axis=1, keepdims=True)
    a1 = jnp.min(jnp.where(masked == l1, iota, _E), axis=1, keepdims=True)
    g = jax.nn.sigmoid(l0 - l1)
    sel0_ref[...] = a0
    sel1_ref[...] = a1
    w0_ref[...] = g
    w1_ref[...] = 1.0 - g


def _route(xf, Wg):
    grid = (_N // _RT,)
    o = jax.ShapeDtypeStruct((_N, 1), jnp.int32)
    of = jax.ShapeDtypeStruct((_N, 1), jnp.float32)
    sel0, sel1, w0, w1 = pl.pallas_call(
        _router_body,
        grid=grid,
        in_specs=[
            pl.BlockSpec((_RT, _DIM), lambda i: (i, 0)),
            pl.BlockSpec((_E, _DIM), lambda i: (0, 0)),
        ],
        out_specs=[
            pl.BlockSpec((_RT, 1), lambda i: (i, 0)),
            pl.BlockSpec((_RT, 1), lambda i: (i, 0)),
            pl.BlockSpec((_RT, 1), lambda i: (i, 0)),
            pl.BlockSpec((_RT, 1), lambda i: (i, 0)),
        ],
        out_shape=[o, o, of, of],
    )(xf, Wg)
    return sel0[:, 0], sel1[:, 0], w0[:, 0], w1[:, 0]


def _dispatch(sel0, sel1, w0, w1):
    sel = jnp.concatenate([sel0, sel1])
    wts = jnp.concatenate([w0, w1])
    ar = jnp.arange(_N, dtype=jnp.int32)
    tok = jnp.concatenate([ar, ar])
    order = jnp.argsort(sel, stable=True)
    sel_s = sel[order]
    counts = jnp.zeros(_E, jnp.int32).at[sel].add(1)
    nb = (counts + _T - 1) // _T
    bend = jnp.cumsum(nb)
    bstart = bend - nb
    cend = jnp.cumsum(counts)
    cstart = cend - counts
    r = jnp.arange(2 * _N, dtype=jnp.int32)
    slot = bstart[sel_s] * _T + (r - cstart[sel_s])
    src_row = jnp.zeros(_NBT, jnp.int32).at[slot].set(tok[order])
    sw = jnp.zeros(_NBT, jnp.float32).at[slot].set(wts[order])
    dest = jnp.zeros(2 * _N, jnp.int32).at[order].set(slot)
    block_expert = jnp.searchsorted(
        bend, jnp.arange(_NB, dtype=jnp.int32), side="right")
    block_expert = jnp.clip(block_expert, 0, _E - 1).astype(jnp.int32)
    return src_row, sw, dest[:_N], dest[_N:], block_expert


def _ffn_body(be_ref, xs_ref, w1_ref, w3_ref, w2_ref, sw_ref, out_ref):
    xb = xs_ref[...]
    a = jax.lax.dot_general(xb, w1_ref[0], (((1,), (1,)), ((), ())),
                            preferred_element_type=jnp.float32)
    b = jax.lax.dot_general(xb, w3_ref[0], (((1,), (1,)), ((), ())),
                            preferred_element_type=jnp.float32)
    h = a * jax.nn.sigmoid(a) * b
    y = jax.lax.dot_general(h, w2_ref[0], (((1,), (1,)), ((), ())),
                            preferred_element_type=jnp.float32)
    out_ref[...] = y * sw_ref[...]


def _grouped_ffn(block_expert, xs, w1, w3, w2, sw):
    grid_spec = pltpu.PrefetchScalarGridSpec(
        num_scalar_prefetch=1,
        grid=(_NB,),
        in_specs=[
            pl.BlockSpec((_T, _DIM), lambda i, be: (i, 0)),
            pl.BlockSpec((1, _HID, _DIM), lambda i, be: (be[i], 0, 0)),
            pl.BlockSpec((1, _HID, _DIM), lambda i, be: (be[i], 0, 0)),
            pl.BlockSpec((1, _DIM, _HID), lambda i, be: (be[i], 0, 0)),
            pl.BlockSpec((_T, 1), lambda i, be: (i, 0)),
        ],
        out_specs=pl.BlockSpec((_T, _DIM), lambda i, be: (i, 0)),
    )
    return pl.pallas_call(
        _ffn_body,
        grid_spec=grid_spec,
        out_shape=jax.ShapeDtypeStruct((_NBT, _DIM), jnp.float32),
    )(block_expert, xs, w1, w3, w2, sw.reshape(_NBT, 1))


def kernel(x, tgt_pad, Wg, w1, w2, w3):
    del tgt_pad
    xf = x.reshape(_N, _DIM)
    sel0, sel1, g0, g1 = _route(xf, Wg)
    src_row, sw, dest0, dest1, block_expert = _dispatch(sel0, sel1, g0, g1)
    xs = xf[src_row]
    ysw = _grouped_ffn(block_expert, xs, w1, w3, w2, sw)
    out = ysw[dest0] + ysw[dest1]
    return out.reshape(_B, _S, _DIM)


# speedup vs baseline: 2.3136x; 1.6001x over previous
import functools

import jax
import jax.numpy as jnp
from jax import lax
from jax.experimental import pallas as pl
from jax.experimental.pallas import tpu as pltpu
from jax.experimental.pallas import tpu_sc as plsc

_B, _S, _DIM = 4, 2048, 1024
_E, _TOPK, _HID = 8, 2, 2048
_N = _B * _S
_T = 256
_NB = (_TOPK * _N) // _T + _E
_NBE = 80
_NBT = _NB * _T
_RT = 1024

_NC = 2
_NSUB = 16
_NW = _NC * _NSUB
_DTOK = _N // _NSUB
_CTOK = _N // _NW
_SGB = 32
_CGB = 16


def _router_body(x_ref, wg_ref, sel0_ref, sel1_ref, w0_ref, w1_ref):
    xb = x_ref[...]
    wg = wg_ref[...]
    logits = jax.lax.dot_general(
        xb, wg, (((1,), (1,)), ((), ())),
        preferred_element_type=jnp.float32)
    iota = jax.lax.broadcasted_iota(jnp.int32, logits.shape, 1)
    l0 = jnp.max(logits, axis=1, keepdims=True)
    a0 = jnp.min(jnp.where(logits == l0, iota, _E), axis=1, keepdims=True)
    masked = jnp.where(iota == a0, -jnp.inf, logits)
    l1 = jnp.max(masked, axis=1, keepdims=True)
    a1 = jnp.min(jnp.where(masked == l1, iota, _E), axis=1, keepdims=True)
    g = jax.nn.sigmoid(l0 - l1)
    sel0_ref[...] = a0
    sel1_ref[...] = a1
    w0_ref[...] = g
    w1_ref[...] = 1.0 - g


def _route(xf, Wg):
    o = jax.ShapeDtypeStruct((_N, 1), jnp.int32)
    of = jax.ShapeDtypeStruct((_N, 1), jnp.float32)
    sel0, sel1, w0, w1 = pl.pallas_call(
        _router_body,
        grid=(_N // _RT,),
        in_specs=[
            pl.BlockSpec((_RT, _DIM), lambda i: (i, 0)),
            pl.BlockSpec((_E, _DIM), lambda i: (0, 0)),
        ],
        out_specs=[
            pl.BlockSpec((_RT, 1), lambda i: (i, 0)),
            pl.BlockSpec((_RT, 1), lambda i: (i, 0)),
            pl.BlockSpec((_RT, 1), lambda i: (i, 0)),
            pl.BlockSpec((_RT, 1), lambda i: (i, 0)),
        ],
        out_shape=[o, o, of, of],
    )(xf, Wg)
    return sel0[:, 0], sel1[:, 0], w0[:, 0], w1[:, 0]


def _dispatch_body(sel0_hbm, sel1_hbm, dest0_hbm, dest1_hbm, be_hbm,
                   stage_hbm, sel0_v, sel1_v, dest0_v, dest1_v, cnt_v, tbl_v,
                   be_v):
    wid = lax.axis_index("s")
    base = wid * _DTOK
    pltpu.sync_copy(sel0_hbm.at[pl.ds(base, _DTOK)], sel0_v)
    pltpu.sync_copy(sel1_hbm.at[pl.ds(base, _DTOK)], sel1_v)
    iota = lax.iota(jnp.int32, 16)
    zero = jnp.zeros(16, jnp.int32)

    def cnt_step(i, cnt):
        v0 = sel0_v[pl.ds(i * 16, 16)]
        v1 = sel1_v[pl.ds(i * 16, 16)]
        for e in range(_E):
            c = (jnp.sum(jnp.where(v0 == e, 1, 0))
                 + jnp.sum(jnp.where(v1 == e, 1, 0)))
            cnt = cnt + jnp.where(iota == e, c, zero)
        return cnt

    cnt_v[...] = lax.fori_loop(0, _DTOK // 16, cnt_step, zero)
    pltpu.sync_copy(cnt_v, stage_hbm.at[wid])
    plsc.subcore_barrier()
    pltpu.sync_copy(stage_hbm, tbl_v)

    def acc_step(j, carry):
        colsum, before = carry
        row = tbl_v[j]
        jb = jnp.broadcast_to(j, (16,))
        wb = jnp.broadcast_to(wid, (16,))
        return colsum + row, before + jnp.where(jb < wb, row, zero)

    colsum, before = lax.fori_loop(0, _NSUB, acc_step, (zero, zero))
    nb = (colsum + (_T - 1)) // _T
    padded = nb * _T
    pstart = plsc.cumsum(padded) - padded
    base_vec0 = pstart + before

    bend = plsc.cumsum(nb)

    @pl.when(wid == 0)
    def _():
        bend_s = [jnp.sum(jnp.where(iota == e, bend, zero)) for e in range(_E)]
        for j in range(_NBE // 16):
            iv = j * 16 + iota
            bexp = jnp.zeros(16, jnp.int32)
            for e in range(_E):
                bexp = bexp + jnp.where(iv >= bend_s[e], 1, 0)
            be_v[pl.ds(j * 16, 16)] = jnp.minimum(bexp, _E - 1)
        pltpu.sync_copy(be_v, be_hbm)

    def walk(i, bv):
        for sel_v, dest_v in ((sel0_v, dest0_v), (sel1_v, dest1_v)):
            v = sel_v[pl.ds(i * 16, 16)]
            dest = jnp.zeros(16, jnp.int32)
            for e in range(_E):
                m = v == e
                mi = jnp.where(m, 1, 0)
                excl = plsc.cumsum(mi) - mi
                be_s = jnp.sum(jnp.where(iota == e, bv, zero))
                dest = jnp.where(m, be_s + excl, dest)
                bv = bv + jnp.where(iota == e, jnp.sum(mi), zero)
            dest_v[pl.ds(i * 16, 16)] = dest
        return bv

    lax.fori_loop(0, _DTOK // 16, walk, base_vec0)
    pltpu.sync_copy(dest0_v, dest0_hbm.at[pl.ds(base, _DTOK)])
    pltpu.sync_copy(dest1_v, dest1_hbm.at[pl.ds(base, _DTOK)])


def _dispatch(sel0, sel1):
    oi = jax.ShapeDtypeStruct((_N,), jnp.int32)
    fn = pl.kernel(
        _dispatch_body,
        mesh=plsc.VectorSubcoreMesh(
            core_axis_name="c", subcore_axis_name="s", num_cores=1),
        compiler_params=pltpu.CompilerParams(needs_layout_passes=False),
        out_type=[oi, oi, jax.ShapeDtypeStruct((_NBE,), jnp.int32),
                  jax.ShapeDtypeStruct((_NSUB, 16), jnp.int32)],
        scratch_types=[
            pltpu.VMEM((_DTOK,), jnp.int32),
            pltpu.VMEM((_DTOK,), jnp.int32),
            pltpu.VMEM((_DTOK,), jnp.int32),
            pltpu.VMEM((_DTOK,), jnp.int32),
            pltpu.VMEM((16,), jnp.int32),
            pltpu.VMEM((_NSUB, 16), jnp.int32),
            pltpu.VMEM((_NBE,), jnp.int32),
        ],
    )
    return fn(sel0, sel1)[:3]


def _scatter_body(x_hbm, d0_hbm, d1_hbm, g0_hbm, g1_hbm, xs_hbm, sw_hbm,
                  idx0_v, idx1_v, wv0_v, wv1_v, rows_a, rows_b, sem, wsem):
    wid = lax.axis_index("s") * _NC + lax.axis_index("c")
    nbatch = _CTOK // _SGB
    ib = wid * nbatch
    pltpu.sync_copy(d0_hbm.at[pl.ds(ib, nbatch)], idx0_v)
    pltpu.sync_copy(d1_hbm.at[pl.ds(ib, nbatch)], idx1_v)
    pltpu.sync_copy(g0_hbm.at[pl.ds(ib, nbatch)], wv0_v)
    pltpu.sync_copy(g1_hbm.at[pl.ds(ib, nbatch)], wv1_v)
    wpend = []
    for b in range(nbatch):
        wpend.append(pltpu.async_copy(wv0_v.at[b], sw_hbm.at[idx0_v.at[b]], wsem))
        wpend.append(pltpu.async_copy(wv1_v.at[b], sw_hbm.at[idx1_v.at[b]], wsem))
    bufs = (rows_a, rows_b)
    pend = []
    for b in range(nbatch):
        buf = bufs[b % 2]
        if b >= 2:
            pend[b - 2][0].wait()
            pend[b - 2][1].wait()
        pltpu.sync_copy(x_hbm.at[pl.ds(wid * _CTOK + b * _SGB, _SGB)], buf)
        h0 = pltpu.async_copy(buf, xs_hbm.at[idx0_v.at[b]], sem)
        h1 = pltpu.async_copy(buf, xs_hbm.at[idx1_v.at[b]], sem)
        pend.append((h0, h1))
    for b in range(nbatch - 2, nbatch):
        pend[b][0].wait()
        pend[b][1].wait()
    for h in wpend:
        h.wait()


def _scatter(xf, d0, d1, g0, g1):
    nbatch = _CTOK // _SGB
    fn = pl.kernel(
        _scatter_body,
        mesh=plsc.VectorSubcoreMesh(
            core_axis_name="c", subcore_axis_name="s", num_cores=_NC),
        out_type=[jax.ShapeDtypeStruct((_NBT, _DIM), jnp.float32),
                  jax.ShapeDtypeStruct((_NBT,), jnp.float32)],
        scratch_types=[
            pltpu.VMEM((nbatch, _SGB), jnp.int32),
            pltpu.VMEM((nbatch, _SGB), jnp.int32),
            pltpu.VMEM((nbatch, _SGB), jnp.float32),
            pltpu.VMEM((nbatch, _SGB), jnp.float32),
            pltpu.VMEM((_SGB, _DIM), jnp.float32),
            pltpu.VMEM((_SGB, _DIM), jnp.float32),
            pltpu.SemaphoreType.DMA,
            pltpu.SemaphoreType.DMA,
        ],
    )
    return fn(xf, d0, d1, g0, g1)


def _ffn_body(be_ref, xs_ref, w1_ref, w3_ref, w2_ref, sw_ref, out_ref):
    xb = xs_ref[...]
    a = jax.lax.dot_general(xb, w1_ref[0], (((1,), (1,)), ((), ())),
                            preferred_element_type=jnp.float32)
    b = jax.lax.dot_general(xb, w3_ref[0], (((1,), (1,)), ((), ())),
                            preferred_element_type=jnp.float32)
    h = a * jax.nn.sigmoid(a) * b
    y = jax.lax.dot_general(h, w2_ref[0], (((1,), (1,)), ((), ())),
                            preferred_element_type=jnp.float32)
    out_ref[...] = y * sw_ref[...]


def _grouped_ffn(block_expert, xs, w1, w3, w2, sw):
    grid_spec = pltpu.PrefetchScalarGridSpec(
        num_scalar_prefetch=1,
        grid=(_NB,),
        in_specs=[
            pl.BlockSpec((_T, _DIM), lambda i, be: (i, 0)),
            pl.BlockSpec((1, _HID, _DIM), lambda i, be: (be[i], 0, 0)),
            pl.BlockSpec((1, _HID, _DIM), lambda i, be: (be[i], 0, 0)),
            pl.BlockSpec((1, _DIM, _HID), lambda i, be: (be[i], 0, 0)),
            pl.BlockSpec((_T, 1), lambda i, be: (i, 0)),
        ],
        out_specs=pl.BlockSpec((_T, _DIM), lambda i, be: (i, 0)),
    )
    return pl.pallas_call(
        _ffn_body,
        grid_spec=grid_spec,
        out_shape=jax.ShapeDtypeStruct((_NBT, _DIM), jnp.float32),
    )(block_expert, xs, w1, w3, w2, sw.reshape(_NBT, 1))


def _combine_body(ys_hbm, d0_hbm, d1_hbm, out_hbm,
                  idx0_v, idx1_v, r0_a, r1_a, r0_b, r1_b, gsem):
    wid = lax.axis_index("s") * _NC + lax.axis_index("c")
    nbatch = _CTOK // _CGB
    ib = wid * nbatch
    pltpu.sync_copy(d0_hbm.at[pl.ds(ib, nbatch)], idx0_v)
    pltpu.sync_copy(d1_hbm.at[pl.ds(ib, nbatch)], idx1_v)
    bufs = ((r0_a, r1_a), (r0_b, r1_b))

    def issue(b):
        buf = bufs[b % 2]
        h0 = pltpu.async_copy(ys_hbm.at[idx0_v.at[b]], buf[0], gsem)
        h1 = pltpu.async_copy(ys_hbm.at[idx1_v.at[b]], buf[1], gsem)
        return h0, h1

    pend = [issue(0), issue(1)]
    for b in range(nbatch):
        buf = bufs[b % 2]
        pend[b][0].wait()
        pend[b][1].wait()

        def add_row(r, carry, buf=buf):
            for c in range(_DIM // 16):
                sl = pl.ds(c * 16, 16)
                buf[0][r, sl] = buf[0][r, sl] + buf[1][r, sl]
            return carry

        lax.fori_loop(0, _CGB, add_row, 0)
        pltpu.sync_copy(buf[0], out_hbm.at[pl.ds((ib + b) * _CGB, _CGB)])
        if b + 2 < nbatch:
            pend.append(issue(b + 2))


def _combine(ys, d0, d1):
    nbatch = _CTOK // _CGB
    fn = pl.kernel(
        _combine_body,
        mesh=plsc.VectorSubcoreMesh(
            core_axis_name="c", subcore_axis_name="s", num_cores=_NC),
        out_type=jax.ShapeDtypeStruct((_N, _DIM), jnp.float32),
        scratch_types=[
            pltpu.VMEM((nbatch, _CGB), jnp.int32),
            pltpu.VMEM((nbatch, _CGB), jnp.int32),
            pltpu.VMEM((_CGB, _DIM), jnp.float32),
            pltpu.VMEM((_CGB, _DIM), jnp.float32),
            pltpu.VMEM((_CGB, _DIM), jnp.float32),
            pltpu.VMEM((_CGB, _DIM), jnp.float32),
            pltpu.SemaphoreType.DMA,
        ],
    )
    return fn(ys, d0, d1)


def kernel(x, tgt_pad, Wg, w1, w2, w3):
    del tgt_pad
    xf = x.reshape(_N, _DIM)
    sel0, sel1, g0, g1 = _route(xf, Wg)
    dest0, dest1, be80 = _dispatch(sel0, sel1)
    xs, sw = _scatter(xf,
                      dest0.reshape(_N // _SGB, _SGB),
                      dest1.reshape(_N // _SGB, _SGB),
                      g0.reshape(_N // _SGB, _SGB),
                      g1.reshape(_N // _SGB, _SGB))
    ysw = _grouped_ffn(be80[:_NB], xs, w1, w3, w2, sw)
    out = _combine(ysw,
                   dest0.reshape(_N // _CGB, _CGB),
                   dest1.reshape(_N // _CGB, _CGB))
    return out.reshape(_B, _S, _DIM)
